# baseline (speedup 1.0000x reference)
import jax
import jax.numpy as jnp
from jax import lax
from jax.experimental import pallas as pl
from jax.experimental.pallas import tpu as pltpu

_DIM = 4
_NHID = 128
_NLAYERS = 3
_RADIUS = 0.35
_N = 4096
_NB = 8
_BT = 128
_NT = _N // _BT
_DSUB = 16
_NSUB = _BT // _DSUB
_NS = _N // _NB


def _mm(a, b):
    return lax.dot_general(a, b, (((1,), (1,)), ((), ())),
                           preferred_element_type=jnp.float32)


def _mmn(a, b):
    return lax.dot_general(a, b, (((1,), (0,)), ((), ())),
                           preferred_element_type=jnp.float32)


def _net_body(lob_ref, hib_ref,
              X_ref, batchc_ref, batch2d_ref, W_enc_ref, b_enc_ref,
              W1a_ref, W1b_ref, bm1_ref, Wm2_ref, bm2_ref,
              Wu1a_ref, Wu1b_ref, bu1_ref, Wu2_ref, bu2_ref,
              W_dec_ref, b_dec_ref,
              out_ref, loss_ref,
              h_ref, A_ref, B_ref, h2_ref, sqc_ref, sqm_ref,
              mask_ref, agg_ref, s1_ref, s2_ref, cnt_ref):
    f32 = jnp.float32
    r2 = jnp.float32(_RADIUS * _RADIUS)

    def enc_tile(t, c):
        xt = X_ref[pl.ds(t * _BT, _BT), :]
        sq = jnp.sum(xt * xt, axis=1, keepdims=True)
        sqc_ref[pl.ds(t * _BT, _BT), :] = sq
        h_ref[pl.ds(t * _BT, _BT), :] = _mm(xt, W_enc_ref[...]) + b_enc_ref[...]
        return c
    lax.fori_loop(0, _NT, enc_tile, 0)
    sqm_ref[...] = sqc_ref[...].reshape(_NT, _BT)

    iota_g = lax.broadcasted_iota(jnp.int32, (_NB, 1), 0)
    iota_gr = lax.broadcasted_iota(jnp.int32, (1, _NB), 1)
    eye_sub = (lax.broadcasted_iota(jnp.int32, (_DSUB, _DSUB), 0)
               == lax.broadcasted_iota(jnp.int32, (_DSUB, _DSUB), 1)
               ).astype(f32)

    for l in range(_NLAYERS):
        W2 = Wm2_ref[l]
        b2 = bm2_ref[l]
        W1a = W1a_ref[l]
        W1b = W1b_ref[l]
        b1 = bm1_ref[l]
        Wu1a = Wu1a_ref[l]
        Wu1b = Wu1b_ref[l]
        bu1 = bu1_ref[l]
        Wu2 = Wu2_ref[l]
        bu2 = bu2_ref[l]

        def ab_tile(t, c, W1a=W1a, W1b=W1b, b1=b1):
            ht = h_ref[pl.ds(t * _BT, _BT), :]
            A_ref[pl.ds(t * _BT, _BT), :] = _mm(ht, W1a) + b1
            B_ref[pl.ds(t * _BT, _BT), :] = _mm(ht, W1b)
            return c
        lax.fori_loop(0, _NT, ab_tile, 0)

        def dst_tile(t, carry, W2=W2, b2=b2, Wu1a=Wu1a, Wu1b=Wu1b,
                     bu1=bu1, Wu2=Wu2, bu2=bu2):
            d0 = t * _BT
            lob = lob_ref[t]
            hib = hib_ref[t]
            xd = X_ref[pl.ds(d0, _BT), :]
            sqd = sqc_ref[pl.ds(d0, _BT), :]
            bd = batchc_ref[pl.ds(d0, _BT), :]
            agg_ref[...] = jnp.zeros((_BT, _NHID), f32)

            def src_blk(s, c2, W2=W2, b2=b2, xd=xd, sqd=sqd, bd=bd, d0=d0):
                s0 = s * _BT
                xs = X_ref[pl.ds(s0, _BT), :]
                d2 = sqd + sqm_ref[pl.ds(s, 1), :] - 2.0 * _mm(xd, xs)
                bs = batch2d_ref[pl.ds(s, 1), :]
                mask_ref[...] = ((d2 <= r2) & (bd == bs)).astype(f32)
                bsrc = B_ref[pl.ds(s0, _BT), :]

                def sub(k, c3, bsrc=bsrc, W2=W2, b2=b2, d0=d0):
                    r0 = k * _DSUB
                    a_sub = A_ref[pl.ds(d0 + r0, _DSUB), :]
                    m1 = jnp.maximum(a_sub[:, None, :] + bsrc[None, :, :],
                                     0.0)
                    m2 = jnp.maximum(
                        lax.dot_general(m1.reshape(_DSUB * _BT, _NHID), W2,
                                        (((1,), (1,)), ((), ())),
                                        precision=lax.Precision.HIGHEST,
                                        preferred_element_type=jnp.float32)
                        + b2, 0.0)
                    mks = mask_ref[pl.ds(r0, _DSUB), :]
                    m3 = m2.reshape(_DSUB, _BT, _NHID) * mks[:, :, None]
                    part = jnp.sum(m3, axis=1)
                    agg_ref[pl.ds(r0, _DSUB), :] = (
                        agg_ref[pl.ds(r0, _DSUB), :] + part)
                    return c3
                lax.fori_loop(0, _NSUB, sub, 0)
                return c2
            lax.fori_loop(lob, hib, src_blk, 0)

            agg = agg_ref[...]
            hd = h_ref[pl.ds(d0, _BT), :]
            u = jnp.maximum(_mm(hd, Wu1a) + _mm(agg, Wu1b) + bu1, 0.0)
            h2_ref[pl.ds(d0, _BT), :] = jnp.maximum(_mm(u, Wu2) + bu2, 0.0)
            return carry
        lax.fori_loop(0, _NT, dst_tile, 0)

        s1_ref[...] = jnp.zeros((_NB, _NHID), f32)
        s2_ref[...] = jnp.zeros((_NB, _NHID), f32)
        cnt_ref[...] = jnp.zeros((_NB, 1), f32)

        def stat_tile(t, c):
            h2t = h2_ref[pl.ds(t * _BT, _BT), :]
            brow = batch2d_ref[pl.ds(t, 1), :]
            oh = (iota_g == brow).astype(f32)
            s1_ref[...] = s1_ref[...] + _mmn(oh, h2t)
            s2_ref[...] = s2_ref[...] + _mmn(oh, h2t * h2t)
            cnt_ref[...] = cnt_ref[...] + jnp.sum(oh, axis=1, keepdims=True)
            return c
        lax.fori_loop(0, _NT, stat_tile, 0)

        cnt = cnt_ref[...]
        mu = s1_ref[...] / cnt
        va = s2_ref[...] / cnt - mu * mu

        def norm_tile(t, c, mu=mu, va=va):
            h2t = h2_ref[pl.ds(t * _BT, _BT), :]
            bcol = batchc_ref[pl.ds(t * _BT, _BT), :]
            ohc = (bcol == iota_gr).astype(f32)
            mug = _mmn(ohc, mu)
            vag = _mmn(ohc, va)
            h_ref[pl.ds(t * _BT, _BT), :] = (h2t - mug) / jnp.sqrt(vag + 1e-5)
            return c
        lax.fori_loop(0, _NT, norm_tile, 0)

    def dec_tile(t, c):
        ht = h_ref[pl.ds(t * _BT, _BT), :]
        out_ref[pl.ds(t * _BT, _BT), :] = jax.nn.sigmoid(
            _mm(ht, W_dec_ref[...]) + b_dec_ref[...])
        return c
    lax.fori_loop(0, _NT, dec_tile, 0)

    iota_d = lax.broadcasted_iota(jnp.int32, (1, _DIM), 1)

    def disc_graph(g, total):
        x = out_ref[pl.ds(g * _NS, _NS), :]
        om = 1.0 - x * x
        p1 = om[:, 0:1] * om[:, 1:2] * om[:, 2:3] * om[:, 3:4]
        sum1 = jnp.sum(p1)
        accp = jnp.ones((_NS, _NS), jnp.float32)
        for d in range(_DIM):
            ed = (iota_d == d).astype(jnp.float32)
            row_d = _mm(ed, x)
            col_d = x[:, d:d + 1]
            accp = accp * (1.0 - jnp.maximum(col_d, row_d))
        sum2 = jnp.sum(accp)
        disc = jnp.sqrt(3.0 ** (-_DIM)
                        - (1.0 / _NS) * (2.0 ** (1 - _DIM)) * sum1
                        + (1.0 / (_NS * _NS)) * sum2)
        return total + disc
    total = lax.fori_loop(0, _NB, disc_graph,
                          jnp.zeros((1, 1), jnp.float32))
    loss_ref[...] = total / _NB


def kernel(X, batch, W_enc, b_enc, Wm1, bm1, Wm2, bm2, Wu1, bu1, Wu2, bu2,
           W_dec, b_dec):
    f32 = jnp.float32
    batch = batch.astype(jnp.int32)
    batchc = batch.reshape(_N, 1)
    batch2d = batch.reshape(_NT, _BT)
    bmin = batch2d[:, 0]
    bmax = batch2d[:, -1]
    lo = jnp.searchsorted(batch, bmin, side="left").astype(jnp.int32)
    hi = jnp.searchsorted(batch, bmax, side="right").astype(jnp.int32)
    lob = lo // _BT
    hib = (hi + _BT - 1) // _BT

    W1a = Wm1[:, :, :_NHID]
    W1b = Wm1[:, :, _NHID:]
    Wm2b = Wm2.astype(jnp.bfloat16)
    Wu1a = Wu1[:, :, :_NHID]
    Wu1b = Wu1[:, :, _NHID:]
    bm1r = bm1.reshape(_NLAYERS, 1, _NHID)
    bu1r = bu1.reshape(_NLAYERS, 1, _NHID)
    bm2r = bm2.reshape(_NLAYERS, 1, _NHID)
    bu2r = bu2.reshape(_NLAYERS, 1, _NHID)
    b_encr = b_enc.reshape(1, _NHID)
    b_decr = b_dec.reshape(1, _DIM)

    smem = pl.BlockSpec(memory_space=pltpu.MemorySpace.SMEM)
    vmem = pl.BlockSpec(memory_space=pltpu.MemorySpace.VMEM)

    out, loss = pl.pallas_call(
        _net_body,
        in_specs=[smem, smem] + [vmem] * 17,
        out_shape=[
            jax.ShapeDtypeStruct((_N, _DIM), f32),
            jax.ShapeDtypeStruct((1, 1), f32),
        ],
        scratch_shapes=[
            pltpu.VMEM((_N, _NHID), f32),
            pltpu.VMEM((_N, _NHID), f32),
            pltpu.VMEM((_N, _NHID), f32),
            pltpu.VMEM((_N, _NHID), f32),
            pltpu.VMEM((_N, 1), f32),
            pltpu.VMEM((_NT, _BT), f32),
            pltpu.VMEM((_BT, _BT), f32),
            pltpu.VMEM((_BT, _NHID), f32),
            pltpu.VMEM((_NB, _NHID), f32),
            pltpu.VMEM((_NB, _NHID), f32),
            pltpu.VMEM((_NB, 1), f32),
        ],
    )(lob, hib,
      X, batchc, batch2d, W_enc, b_encr,
      W1a, W1b, bm1r, Wm2, bm2r,
      Wu1a, Wu1b, bu1r, Wu2, bu2r,
      W_dec, b_decr)
    return (loss[0, 0], out.reshape(_NB, _NS, _DIM))


# speedup vs baseline: 2.7710x; 2.7710x over previous
import jax
import jax.numpy as jnp
from jax import lax
from jax.experimental import pallas as pl
from jax.experimental.pallas import tpu as pltpu

_DIM = 4
_NHID = 128
_NLAYERS = 3
_RADIUS = 0.35
_N = 4096
_NB = 8
_BT = 128
_NT = _N // _BT
_DSUB = 16
_NSUB = _BT // _DSUB
_NS = _N // _NB


def _mm(a, b):
    return lax.dot_general(a, b, (((1,), (1,)), ((), ())),
                           preferred_element_type=jnp.float32)


def _mmn(a, b):
    return lax.dot_general(a, b, (((1,), (0,)), ((), ())),
                           preferred_element_type=jnp.float32)


def _net_body(lob_ref, hib_ref,
              X_ref, batchc_ref, batch2d_ref, W_enc_ref, b_enc_ref,
              W1a_ref, W1b_ref, bm1_ref, Wm2_ref, bm2_ref,
              Wu1a_ref, Wu1b_ref, bu1_ref, Wu2_ref, bu2_ref,
              W_dec_ref, b_dec_ref,
              out_ref, loss_ref,
              h_ref, A_ref, B_ref, h2_ref, sqc_ref, sqm_ref,
              mask_ref, agg_ref, s1_ref, s2_ref, cnt_ref):
    f32 = jnp.float32
    r2 = jnp.float32(_RADIUS * _RADIUS)

    def enc_tile(t, c):
        xt = X_ref[pl.ds(t * _BT, _BT), :]
        sq = jnp.sum(xt * xt, axis=1, keepdims=True)
        sqc_ref[pl.ds(t * _BT, _BT), :] = sq
        h_ref[pl.ds(t * _BT, _BT), :] = _mm(xt, W_enc_ref[...]) + b_enc_ref[...]
        return c
    lax.fori_loop(0, _NT, enc_tile, 0)
    sqm_ref[...] = sqc_ref[...].reshape(_NT, _BT)

    iota_g = lax.broadcasted_iota(jnp.int32, (_NB, 1), 0)
    iota_gr = lax.broadcasted_iota(jnp.int32, (1, _NB), 1)
    eye_sub = (lax.broadcasted_iota(jnp.int32, (_DSUB, _DSUB), 0)
               == lax.broadcasted_iota(jnp.int32, (_DSUB, _DSUB), 1)
               ).astype(f32)

    for l in range(_NLAYERS):
        W2 = Wm2_ref[l]
        b2 = bm2_ref[l]
        W1a = W1a_ref[l]
        W1b = W1b_ref[l]
        b1 = bm1_ref[l]
        Wu1a = Wu1a_ref[l]
        Wu1b = Wu1b_ref[l]
        bu1 = bu1_ref[l]
        Wu2 = Wu2_ref[l]
        bu2 = bu2_ref[l]

        def ab_tile(t, c, W1a=W1a, W1b=W1b, b1=b1):
            ht = h_ref[pl.ds(t * _BT, _BT), :]
            A_ref[pl.ds(t * _BT, _BT), :] = _mm(ht, W1a) + b1
            B_ref[pl.ds(t * _BT, _BT), :] = _mm(ht, W1b)
            return c
        lax.fori_loop(0, _NT, ab_tile, 0)

        def dst_tile(t, carry, W2=W2, b2=b2, Wu1a=Wu1a, Wu1b=Wu1b,
                     bu1=bu1, Wu2=Wu2, bu2=bu2):
            d0 = t * _BT
            lob = lob_ref[t]
            hib = hib_ref[t]
            xd = X_ref[pl.ds(d0, _BT), :]
            sqd = sqc_ref[pl.ds(d0, _BT), :]
            bd = batchc_ref[pl.ds(d0, _BT), :]
            agg_ref[...] = jnp.zeros((_BT, _NHID), f32)

            def src_blk(s, c2, W2=W2, b2=b2, xd=xd, sqd=sqd, bd=bd, d0=d0):
                s0 = s * _BT
                xs = X_ref[pl.ds(s0, _BT), :]
                d2 = sqd + sqm_ref[pl.ds(s, 1), :] - 2.0 * _mm(xd, xs)
                bs = batch2d_ref[pl.ds(s, 1), :]
                mask_ref[...] = ((d2 <= r2) & (bd == bs)).astype(f32)
                bsrc = B_ref[pl.ds(s0, _BT), :]

                def strip(k, bsrc=bsrc, W2=W2, b2=b2, d0=d0):
                    r0 = k * _DSUB
                    a_sub = A_ref[pl.ds(d0 + r0, _DSUB), :]
                    m1 = jnp.maximum(
                        (a_sub[:, None, :] + bsrc[None, :, :]
                         ).astype(jnp.bfloat16),
                        jnp.bfloat16(0.0))
                    m2 = jnp.maximum(
                        _mm(m1.reshape(_DSUB * _BT, _NHID), W2) + b2, 0.0)
                    mks = mask_ref[pl.ds(r0, _DSUB), :]
                    m3 = m2.reshape(_DSUB, _BT, _NHID) * mks[:, :, None]
                    part = jnp.sum(m3, axis=1)
                    agg_ref[pl.ds(r0, _DSUB), :] = (
                        agg_ref[pl.ds(r0, _DSUB), :] + part)

                def sub(i, c3):
                    strip(2 * i)
                    strip(2 * i + 1)
                    return c3
                lax.fori_loop(0, _NSUB // 2, sub, 0)
                return c2
            lax.fori_loop(lob, hib, src_blk, 0)

            agg = agg_ref[...]
            hd = h_ref[pl.ds(d0, _BT), :]
            u = jnp.maximum(_mm(hd, Wu1a) + _mm(agg, Wu1b) + bu1, 0.0)
            h2_ref[pl.ds(d0, _BT), :] = jnp.maximum(_mm(u, Wu2) + bu2, 0.0)
            return carry
        lax.fori_loop(0, _NT, dst_tile, 0)

        s1_ref[...] = jnp.zeros((_NB, _NHID), f32)
        s2_ref[...] = jnp.zeros((_NB, _NHID), f32)
        cnt_ref[...] = jnp.zeros((_NB, 1), f32)

        def stat_tile(t, c):
            h2t = h2_ref[pl.ds(t * _BT, _BT), :]
            brow = batch2d_ref[pl.ds(t, 1), :]
            oh = (iota_g == brow).astype(f32)
            s1_ref[...] = s1_ref[...] + _mmn(oh, h2t)
            s2_ref[...] = s2_ref[...] + _mmn(oh, h2t * h2t)
            cnt_ref[...] = cnt_ref[...] + jnp.sum(oh, axis=1, keepdims=True)
            return c
        lax.fori_loop(0, _NT, stat_tile, 0)

        cnt = cnt_ref[...]
        mu = s1_ref[...] / cnt
        va = s2_ref[...] / cnt - mu * mu

        def norm_tile(t, c, mu=mu, va=va):
            h2t = h2_ref[pl.ds(t * _BT, _BT), :]
            bcol = batchc_ref[pl.ds(t * _BT, _BT), :]
            ohc = (bcol == iota_gr).astype(f32)
            mug = _mmn(ohc, mu)
            vag = _mmn(ohc, va)
            h_ref[pl.ds(t * _BT, _BT), :] = (h2t - mug) / jnp.sqrt(vag + 1e-5)
            return c
        lax.fori_loop(0, _NT, norm_tile, 0)

    def dec_tile(t, c):
        ht = h_ref[pl.ds(t * _BT, _BT), :]
        out_ref[pl.ds(t * _BT, _BT), :] = jax.nn.sigmoid(
            _mm(ht, W_dec_ref[...]) + b_dec_ref[...])
        return c
    lax.fori_loop(0, _NT, dec_tile, 0)

    iota_d = lax.broadcasted_iota(jnp.int32, (1, _DIM), 1)

    def disc_graph(g, total):
        x = out_ref[pl.ds(g * _NS, _NS), :]
        om = 1.0 - x * x
        p1 = om[:, 0:1] * om[:, 1:2] * om[:, 2:3] * om[:, 3:4]
        sum1 = jnp.sum(p1)
        accp = jnp.ones((_NS, _NS), jnp.float32)
        for d in range(_DIM):
            ed = (iota_d == d).astype(jnp.float32)
            row_d = _mm(ed, x)
            col_d = x[:, d:d + 1]
            accp = accp * (1.0 - jnp.maximum(col_d, row_d))
        sum2 = jnp.sum(accp)
        disc = jnp.sqrt(3.0 ** (-_DIM)
                        - (1.0 / _NS) * (2.0 ** (1 - _DIM)) * sum1
                        + (1.0 / (_NS * _NS)) * sum2)
        return total + disc
    total = lax.fori_loop(0, _NB, disc_graph,
                          jnp.zeros((1, 1), jnp.float32))
    loss_ref[...] = total / _NB


def kernel(X, batch, W_enc, b_enc, Wm1, bm1, Wm2, bm2, Wu1, bu1, Wu2, bu2,
           W_dec, b_dec):
    f32 = jnp.float32
    batch = batch.astype(jnp.int32)
    batchc = batch.reshape(_N, 1)
    batch2d = batch.reshape(_NT, _BT)
    bmin = batch2d[:, 0]
    bmax = batch2d[:, -1]
    lo = jnp.searchsorted(batch, bmin, side="left").astype(jnp.int32)
    hi = jnp.searchsorted(batch, bmax, side="right").astype(jnp.int32)
    lob = lo // _BT
    hib = (hi + _BT - 1) // _BT

    W1a = Wm1[:, :, :_NHID]
    W1b = Wm1[:, :, _NHID:]
    Wm2b = Wm2.astype(jnp.bfloat16)
    Wu1a = Wu1[:, :, :_NHID]
    Wu1b = Wu1[:, :, _NHID:]
    bm1r = bm1.reshape(_NLAYERS, 1, _NHID)
    bu1r = bu1.reshape(_NLAYERS, 1, _NHID)
    bm2r = bm2.reshape(_NLAYERS, 1, _NHID)
    bu2r = bu2.reshape(_NLAYERS, 1, _NHID)
    b_encr = b_enc.reshape(1, _NHID)
    b_decr = b_dec.reshape(1, _DIM)

    smem = pl.BlockSpec(memory_space=pltpu.MemorySpace.SMEM)
    vmem = pl.BlockSpec(memory_space=pltpu.MemorySpace.VMEM)

    out, loss = pl.pallas_call(
        _net_body,
        in_specs=[smem, smem] + [vmem] * 17,
        out_shape=[
            jax.ShapeDtypeStruct((_N, _DIM), f32),
            jax.ShapeDtypeStruct((1, 1), f32),
        ],
        scratch_shapes=[
            pltpu.VMEM((_N, _NHID), f32),
            pltpu.VMEM((_N, _NHID), f32),
            pltpu.VMEM((_N, _NHID), f32),
            pltpu.VMEM((_N, _NHID), f32),
            pltpu.VMEM((_N, 1), f32),
            pltpu.VMEM((_NT, _BT), f32),
            pltpu.VMEM((_BT, _BT), f32),
            pltpu.VMEM((_BT, _NHID), f32),
            pltpu.VMEM((_NB, _NHID), f32),
            pltpu.VMEM((_NB, _NHID), f32),
            pltpu.VMEM((_NB, 1), f32),
        ],
    )(lob, hib,
      X, batchc, batch2d, W_enc, b_encr,
      W1a, W1b, bm1r, Wm2b, bm2r,
      Wu1a, Wu1b, bu1r, Wu2, bu2r,
      W_dec, b_decr)
    return (loss[0, 0], out.reshape(_NB, _NS, _DIM))


# speedup vs baseline: 2.9704x; 1.0719x over previous
import jax
import jax.numpy as jnp
from jax import lax
from jax.experimental import pallas as pl
from jax.experimental.pallas import tpu as pltpu

_DIM = 4
_NHID = 128
_NLAYERS = 3
_RADIUS = 0.35
_N = 4096
_NB = 8
_BT = 128
_NT = _N // _BT
_DSUB = 16
_NSUB = _BT // _DSUB
_NS = _N // _NB


def _mm(a, b):
    return lax.dot_general(a, b, (((1,), (1,)), ((), ())),
                           preferred_element_type=jnp.float32)


def _mmn(a, b):
    return lax.dot_general(a, b, (((1,), (0,)), ((), ())),
                           preferred_element_type=jnp.float32)


def _net_body(lob_ref, hib_ref,
              X_ref, batchc_ref, batch2d_ref, W_enc_ref, b_enc_ref,
              W1a_ref, W1b_ref, bm1_ref, Wm2_ref, bm2_ref,
              Wu1a_ref, Wu1b_ref, bu1_ref, Wu2_ref, bu2_ref,
              W_dec_ref, b_dec_ref,
              out_ref, loss_ref,
              h_ref, A_ref, B_ref, h2_ref, sqc_ref, sqm_ref,
              mask_ref, agg_ref, s1_ref, s2_ref, cnt_ref):
    f32 = jnp.float32
    r2 = jnp.float32(_RADIUS * _RADIUS)

    def enc_tile(t, c):
        xt = X_ref[pl.ds(t * _BT, _BT), :]
        sq = jnp.sum(xt * xt, axis=1, keepdims=True)
        sqc_ref[pl.ds(t * _BT, _BT), :] = sq
        h_ref[pl.ds(t * _BT, _BT), :] = _mm(xt, W_enc_ref[...]) + b_enc_ref[...]
        return c
    lax.fori_loop(0, _NT, enc_tile, 0)
    sqm_ref[...] = sqc_ref[...].reshape(_NT, _BT)

    iota_g = lax.broadcasted_iota(jnp.int32, (_NB, 1), 0)
    iota_gr = lax.broadcasted_iota(jnp.int32, (1, _NB), 1)
    eye_sub = (lax.broadcasted_iota(jnp.int32, (_DSUB, _DSUB), 0)
               == lax.broadcasted_iota(jnp.int32, (_DSUB, _DSUB), 1)
               ).astype(f32)

    for l in range(_NLAYERS):
        W2 = Wm2_ref[l]
        b2 = bm2_ref[l]
        W1a = W1a_ref[l]
        W1b = W1b_ref[l]
        b1 = bm1_ref[l]
        Wu1a = Wu1a_ref[l]
        Wu1b = Wu1b_ref[l]
        bu1 = bu1_ref[l]
        Wu2 = Wu2_ref[l]
        bu2 = bu2_ref[l]

        def ab_tile(t, c, W1a=W1a, W1b=W1b, b1=b1):
            ht = h_ref[pl.ds(t * _BT, _BT), :]
            A_ref[pl.ds(t * _BT, _BT), :] = _mm(ht, W1a) + b1
            B_ref[pl.ds(t * _BT, _BT), :] = _mm(ht, W1b)
            return c
        lax.fori_loop(0, _NT, ab_tile, 0)

        def dst_tile(t, carry, W2=W2, b2=b2, Wu1a=Wu1a, Wu1b=Wu1b,
                     bu1=bu1, Wu2=Wu2, bu2=bu2):
            d0 = t * _BT
            lob = lob_ref[t]
            hib = hib_ref[t]
            xd = X_ref[pl.ds(d0, _BT), :]
            sqd = sqc_ref[pl.ds(d0, _BT), :]
            bd = batchc_ref[pl.ds(d0, _BT), :]
            agg_ref[...] = jnp.zeros((_BT, _NHID), f32)

            def src_blk(s, c2, W2=W2, b2=b2, xd=xd, sqd=sqd, bd=bd, d0=d0):
                s0 = s * _BT
                xs = X_ref[pl.ds(s0, _BT), :]
                d2 = sqd + sqm_ref[pl.ds(s, 1), :] - 2.0 * _mm(xd, xs)
                bs = batch2d_ref[pl.ds(s, 1), :]
                mask_ref[...] = ((d2 <= r2) & (bd == bs)).astype(f32)
                bsrc = B_ref[pl.ds(s0, _BT), :]

                def strip(k, bsrc=bsrc, W2=W2, b2=b2, d0=d0):
                    r0 = k * _DSUB
                    a_sub = A_ref[pl.ds(d0 + r0, _DSUB), :]
                    m1 = jnp.maximum(
                        (a_sub[:, None, :] + bsrc[None, :, :]
                         ).astype(jnp.bfloat16),
                        jnp.bfloat16(0.0))
                    m2 = jnp.maximum(
                        _mm(m1.reshape(_DSUB * _BT, _NHID), W2) + b2, 0.0)
                    mks = mask_ref[pl.ds(r0, _DSUB), :]
                    m3 = m2.reshape(_DSUB, _BT, _NHID) * mks[:, :, None]
                    part = jnp.sum(m3, axis=1)
                    agg_ref[pl.ds(r0, _DSUB), :] = (
                        agg_ref[pl.ds(r0, _DSUB), :] + part)

                def sub(i, c3):
                    strip(4 * i)
                    strip(4 * i + 1)
                    strip(4 * i + 2)
                    strip(4 * i + 3)
                    return c3
                lax.fori_loop(0, _NSUB // 4, sub, 0)
                return c2
            lax.fori_loop(lob, hib, src_blk, 0)

            agg = agg_ref[...]
            hd = h_ref[pl.ds(d0, _BT), :]
            u = jnp.maximum(_mm(hd, Wu1a) + _mm(agg, Wu1b) + bu1, 0.0)
            h2_ref[pl.ds(d0, _BT), :] = jnp.maximum(_mm(u, Wu2) + bu2, 0.0)
            return carry
        lax.fori_loop(0, _NT, dst_tile, 0)

        s1_ref[...] = jnp.zeros((_NB, _NHID), f32)
        s2_ref[...] = jnp.zeros((_NB, _NHID), f32)
        cnt_ref[...] = jnp.zeros((_NB, 1), f32)

        def stat_tile(t, c):
            h2t = h2_ref[pl.ds(t * _BT, _BT), :]
            brow = batch2d_ref[pl.ds(t, 1), :]
            oh = (iota_g == brow).astype(f32)
            s1_ref[...] = s1_ref[...] + _mmn(oh, h2t)
            s2_ref[...] = s2_ref[...] + _mmn(oh, h2t * h2t)
            cnt_ref[...] = cnt_ref[...] + jnp.sum(oh, axis=1, keepdims=True)
            return c
        lax.fori_loop(0, _NT, stat_tile, 0)

        cnt = cnt_ref[...]
        mu = s1_ref[...] / cnt
        va = s2_ref[...] / cnt - mu * mu

        def norm_tile(t, c, mu=mu, va=va):
            h2t = h2_ref[pl.ds(t * _BT, _BT), :]
            bcol = batchc_ref[pl.ds(t * _BT, _BT), :]
            ohc = (bcol == iota_gr).astype(f32)
            mug = _mmn(ohc, mu)
            vag = _mmn(ohc, va)
            h_ref[pl.ds(t * _BT, _BT), :] = (h2t - mug) / jnp.sqrt(vag + 1e-5)
            return c
        lax.fori_loop(0, _NT, norm_tile, 0)

    def dec_tile(t, c):
        ht = h_ref[pl.ds(t * _BT, _BT), :]
        out_ref[pl.ds(t * _BT, _BT), :] = jax.nn.sigmoid(
            _mm(ht, W_dec_ref[...]) + b_dec_ref[...])
        return c
    lax.fori_loop(0, _NT, dec_tile, 0)

    iota_d = lax.broadcasted_iota(jnp.int32, (1, _DIM), 1)

    def disc_graph(g, total):
        x = out_ref[pl.ds(g * _NS, _NS), :]
        om = 1.0 - x * x
        p1 = om[:, 0:1] * om[:, 1:2] * om[:, 2:3] * om[:, 3:4]
        sum1 = jnp.sum(p1)
        accp = jnp.ones((_NS, _NS), jnp.float32)
        for d in range(_DIM):
            ed = (iota_d == d).astype(jnp.float32)
            row_d = _mm(ed, x)
            col_d = x[:, d:d + 1]
            accp = accp * (1.0 - jnp.maximum(col_d, row_d))
        sum2 = jnp.sum(accp)
        disc = jnp.sqrt(3.0 ** (-_DIM)
                        - (1.0 / _NS) * (2.0 ** (1 - _DIM)) * sum1
                        + (1.0 / (_NS * _NS)) * sum2)
        return total + disc
    total = lax.fori_loop(0, _NB, disc_graph,
                          jnp.zeros((1, 1), jnp.float32))
    loss_ref[...] = total / _NB


def kernel(X, batch, W_enc, b_enc, Wm1, bm1, Wm2, bm2, Wu1, bu1, Wu2, bu2,
           W_dec, b_dec):
    f32 = jnp.float32
    batch = batch.astype(jnp.int32)
    batchc = batch.reshape(_N, 1)
    batch2d = batch.reshape(_NT, _BT)
    bmin = batch2d[:, 0]
    bmax = batch2d[:, -1]
    lo = jnp.searchsorted(batch, bmin, side="left").astype(jnp.int32)
    hi = jnp.searchsorted(batch, bmax, side="right").astype(jnp.int32)
    lob = lo // _BT
    hib = (hi + _BT - 1) // _BT

    W1a = Wm1[:, :, :_NHID]
    W1b = Wm1[:, :, _NHID:]
    Wm2b = Wm2.astype(jnp.bfloat16)
    Wu1a = Wu1[:, :, :_NHID]
    Wu1b = Wu1[:, :, _NHID:]
    bm1r = bm1.reshape(_NLAYERS, 1, _NHID)
    bu1r = bu1.reshape(_NLAYERS, 1, _NHID)
    bm2r = bm2.reshape(_NLAYERS, 1, _NHID)
    bu2r = bu2.reshape(_NLAYERS, 1, _NHID)
    b_encr = b_enc.reshape(1, _NHID)
    b_decr = b_dec.reshape(1, _DIM)

    smem = pl.BlockSpec(memory_space=pltpu.MemorySpace.SMEM)
    vmem = pl.BlockSpec(memory_space=pltpu.MemorySpace.VMEM)

    out, loss = pl.pallas_call(
        _net_body,
        in_specs=[smem, smem] + [vmem] * 17,
        out_shape=[
            jax.ShapeDtypeStruct((_N, _DIM), f32),
            jax.ShapeDtypeStruct((1, 1), f32),
        ],
        scratch_shapes=[
            pltpu.VMEM((_N, _NHID), f32),
            pltpu.VMEM((_N, _NHID), f32),
            pltpu.VMEM((_N, _NHID), f32),
            pltpu.VMEM((_N, _NHID), f32),
            pltpu.VMEM((_N, 1), f32),
            pltpu.VMEM((_NT, _BT), f32),
            pltpu.VMEM((_BT, _BT), f32),
            pltpu.VMEM((_BT, _NHID), f32),
            pltpu.VMEM((_NB, _NHID), f32),
            pltpu.VMEM((_NB, _NHID), f32),
            pltpu.VMEM((_NB, 1), f32),
        ],
    )(lob, hib,
      X, batchc, batch2d, W_enc, b_encr,
      W1a, W1b, bm1r, Wm2b, bm2r,
      Wu1a, Wu1b, bu1r, Wu2, bu2r,
      W_dec, b_decr)
    return (loss[0, 0], out.reshape(_NB, _NS, _DIM))


# speedup vs baseline: 3.0949x; 1.0419x over previous
import jax
import jax.numpy as jnp
from jax import lax
from jax.experimental import pallas as pl
from jax.experimental.pallas import tpu as pltpu

_DIM = 4
_NHID = 128
_NLAYERS = 3
_RADIUS = 0.35
_N = 4096
_NB = 8
_BT = 128
_NT = _N // _BT
_DSUB = 16
_NSUB = _BT // _DSUB
_NS = _N // _NB


def _mm(a, b):
    return lax.dot_general(a, b, (((1,), (1,)), ((), ())),
                           preferred_element_type=jnp.float32)


def _mmn(a, b):
    return lax.dot_general(a, b, (((1,), (0,)), ((), ())),
                           preferred_element_type=jnp.float32)


def _net_body(lob_ref, hib_ref,
              X_ref, batchc_ref, batch2d_ref, W_enc_ref, b_enc_ref,
              W1a_ref, W1b_ref, bm1_ref, Wm2_ref, bm2_ref,
              Wu1a_ref, Wu1b_ref, bu1_ref, Wu2_ref, bu2_ref,
              W_dec_ref, b_dec_ref,
              out_ref, loss_ref,
              h_ref, A_ref, B_ref, h2_ref, sqc_ref, sqm_ref,
              mask_ref, agg_ref, s1_ref, s2_ref, cnt_ref):
    f32 = jnp.float32
    r2 = jnp.float32(_RADIUS * _RADIUS)

    def enc_tile(t, c):
        xt = X_ref[pl.ds(t * _BT, _BT), :]
        sq = jnp.sum(xt * xt, axis=1, keepdims=True)
        sqc_ref[pl.ds(t * _BT, _BT), :] = sq
        h_ref[pl.ds(t * _BT, _BT), :] = _mm(xt, W_enc_ref[...]) + b_enc_ref[...]
        return c
    lax.fori_loop(0, _NT, enc_tile, 0)
    sqm_ref[...] = sqc_ref[...].reshape(_NT, _BT)

    iota_g = lax.broadcasted_iota(jnp.int32, (_NB, 1), 0)
    iota_gr = lax.broadcasted_iota(jnp.int32, (1, _NB), 1)
    eye_sub = (lax.broadcasted_iota(jnp.int32, (_DSUB, _DSUB), 0)
               == lax.broadcasted_iota(jnp.int32, (_DSUB, _DSUB), 1)
               ).astype(f32)

    for l in range(_NLAYERS):
        W2 = Wm2_ref[l]
        b2 = bm2_ref[l]
        W1a = W1a_ref[l]
        W1b = W1b_ref[l]
        b1 = bm1_ref[l]
        Wu1a = Wu1a_ref[l]
        Wu1b = Wu1b_ref[l]
        bu1 = bu1_ref[l]
        Wu2 = Wu2_ref[l]
        bu2 = bu2_ref[l]

        def ab_tile(t, c, W1a=W1a, W1b=W1b, b1=b1):
            ht = h_ref[pl.ds(t * _BT, _BT), :]
            A_ref[pl.ds(t * _BT, _BT), :] = _mm(ht, W1a) + b1
            B_ref[pl.ds(t * _BT, _BT), :] = _mm(ht, W1b)
            return c
        lax.fori_loop(0, _NT, ab_tile, 0)

        def dst_tile(t, carry, W2=W2, b2=b2, Wu1a=Wu1a, Wu1b=Wu1b,
                     bu1=bu1, Wu2=Wu2, bu2=bu2):
            d0 = t * _BT
            lob = lob_ref[t]
            hib = hib_ref[t]
            xd = X_ref[pl.ds(d0, _BT), :]
            sqd = sqc_ref[pl.ds(d0, _BT), :]
            bd = batchc_ref[pl.ds(d0, _BT), :]
            agg_ref[...] = jnp.zeros((_BT, _NHID), f32)

            def src_blk(s, c2, W2=W2, b2=b2, xd=xd, sqd=sqd, bd=bd, d0=d0):
                s0 = s * _BT
                xs = X_ref[pl.ds(s0, _BT), :]
                d2 = sqd + sqm_ref[pl.ds(s, 1), :] - 2.0 * _mm(xd, xs)
                bs = batch2d_ref[pl.ds(s, 1), :]
                mask_ref[...] = ((d2 <= r2) & (bd == bs)).astype(f32)
                bsrc = B_ref[pl.ds(s0, _BT), :]

                def strip(k, bsrc=bsrc, W2=W2, b2=b2, d0=d0):
                    r0 = k * _DSUB
                    a_sub = A_ref[pl.ds(d0 + r0, _DSUB), :]
                    m1 = jnp.maximum(
                        (a_sub[:, None, :] + bsrc[None, :, :]
                         ).astype(jnp.bfloat16),
                        jnp.bfloat16(0.0))
                    m2 = jnp.maximum(
                        _mm(m1.reshape(_DSUB * _BT, _NHID), W2) + b2, 0.0)
                    mks = mask_ref[pl.ds(r0, _DSUB), :]
                    m3 = m2.reshape(_DSUB, _BT, _NHID) * mks[:, :, None]
                    part = jnp.sum(m3, axis=1)
                    agg_ref[pl.ds(r0, _DSUB), :] = (
                        agg_ref[pl.ds(r0, _DSUB), :] + part)

                for k in range(_NSUB):
                    strip(k)
                return c2
            lax.fori_loop(lob, hib, src_blk, 0)

            agg = agg_ref[...]
            hd = h_ref[pl.ds(d0, _BT), :]
            u = jnp.maximum(_mm(hd, Wu1a) + _mm(agg, Wu1b) + bu1, 0.0)
            h2_ref[pl.ds(d0, _BT), :] = jnp.maximum(_mm(u, Wu2) + bu2, 0.0)
            return carry
        lax.fori_loop(0, _NT, dst_tile, 0)

        s1_ref[...] = jnp.zeros((_NB, _NHID), f32)
        s2_ref[...] = jnp.zeros((_NB, _NHID), f32)
        cnt_ref[...] = jnp.zeros((_NB, 1), f32)

        def stat_tile(t, c):
            h2t = h2_ref[pl.ds(t * _BT, _BT), :]
            brow = batch2d_ref[pl.ds(t, 1), :]
            oh = (iota_g == brow).astype(f32)
            s1_ref[...] = s1_ref[...] + _mmn(oh, h2t)
            s2_ref[...] = s2_ref[...] + _mmn(oh, h2t * h2t)
            cnt_ref[...] = cnt_ref[...] + jnp.sum(oh, axis=1, keepdims=True)
            return c
        lax.fori_loop(0, _NT, stat_tile, 0)

        cnt = cnt_ref[...]
        mu = s1_ref[...] / cnt
        va = s2_ref[...] / cnt - mu * mu

        def norm_tile(t, c, mu=mu, va=va):
            h2t = h2_ref[pl.ds(t * _BT, _BT), :]
            bcol = batchc_ref[pl.ds(t * _BT, _BT), :]
            ohc = (bcol == iota_gr).astype(f32)
            mug = _mmn(ohc, mu)
            vag = _mmn(ohc, va)
            h_ref[pl.ds(t * _BT, _BT), :] = (h2t - mug) / jnp.sqrt(vag + 1e-5)
            return c
        lax.fori_loop(0, _NT, norm_tile, 0)

    def dec_tile(t, c):
        ht = h_ref[pl.ds(t * _BT, _BT), :]
        out_ref[pl.ds(t * _BT, _BT), :] = jax.nn.sigmoid(
            _mm(ht, W_dec_ref[...]) + b_dec_ref[...])
        return c
    lax.fori_loop(0, _NT, dec_tile, 0)

    iota_d = lax.broadcasted_iota(jnp.int32, (1, _DIM), 1)

    def disc_graph(g, total):
        x = out_ref[pl.ds(g * _NS, _NS), :]
        om = 1.0 - x * x
        p1 = om[:, 0:1] * om[:, 1:2] * om[:, 2:3] * om[:, 3:4]
        sum1 = jnp.sum(p1)
        accp = jnp.ones((_NS, _NS), jnp.float32)
        for d in range(_DIM):
            ed = (iota_d == d).astype(jnp.float32)
            row_d = _mm(ed, x)
            col_d = x[:, d:d + 1]
            accp = accp * (1.0 - jnp.maximum(col_d, row_d))
        sum2 = jnp.sum(accp)
        disc = jnp.sqrt(3.0 ** (-_DIM)
                        - (1.0 / _NS) * (2.0 ** (1 - _DIM)) * sum1
                        + (1.0 / (_NS * _NS)) * sum2)
        return total + disc
    total = lax.fori_loop(0, _NB, disc_graph,
                          jnp.zeros((1, 1), jnp.float32))
    loss_ref[...] = total / _NB


def kernel(X, batch, W_enc, b_enc, Wm1, bm1, Wm2, bm2, Wu1, bu1, Wu2, bu2,
           W_dec, b_dec):
    f32 = jnp.float32
    batch = batch.astype(jnp.int32)
    batchc = batch.reshape(_N, 1)
    batch2d = batch.reshape(_NT, _BT)
    bmin = batch2d[:, 0]
    bmax = batch2d[:, -1]
    lo = jnp.searchsorted(batch, bmin, side="left").astype(jnp.int32)
    hi = jnp.searchsorted(batch, bmax, side="right").astype(jnp.int32)
    lob = lo // _BT
    hib = (hi + _BT - 1) // _BT

    W1a = Wm1[:, :, :_NHID]
    W1b = Wm1[:, :, _NHID:]
    Wm2b = Wm2.astype(jnp.bfloat16)
    Wu1a = Wu1[:, :, :_NHID]
    Wu1b = Wu1[:, :, _NHID:]
    bm1r = bm1.reshape(_NLAYERS, 1, _NHID)
    bu1r = bu1.reshape(_NLAYERS, 1, _NHID)
    bm2r = bm2.reshape(_NLAYERS, 1, _NHID)
    bu2r = bu2.reshape(_NLAYERS, 1, _NHID)
    b_encr = b_enc.reshape(1, _NHID)
    b_decr = b_dec.reshape(1, _DIM)

    smem = pl.BlockSpec(memory_space=pltpu.MemorySpace.SMEM)
    vmem = pl.BlockSpec(memory_space=pltpu.MemorySpace.VMEM)

    out, loss = pl.pallas_call(
        _net_body,
        in_specs=[smem, smem] + [vmem] * 17,
        out_shape=[
            jax.ShapeDtypeStruct((_N, _DIM), f32),
            jax.ShapeDtypeStruct((1, 1), f32),
        ],
        scratch_shapes=[
            pltpu.VMEM((_N, _NHID), f32),
            pltpu.VMEM((_N, _NHID), f32),
            pltpu.VMEM((_N, _NHID), f32),
            pltpu.VMEM((_N, _NHID), f32),
            pltpu.VMEM((_N, 1), f32),
            pltpu.VMEM((_NT, _BT), f32),
            pltpu.VMEM((_BT, _BT), f32),
            pltpu.VMEM((_BT, _NHID), f32),
            pltpu.VMEM((_NB, _NHID), f32),
            pltpu.VMEM((_NB, _NHID), f32),
            pltpu.VMEM((_NB, 1), f32),
        ],
    )(lob, hib,
      X, batchc, batch2d, W_enc, b_encr,
      W1a, W1b, bm1r, Wm2b, bm2r,
      Wu1a, Wu1b, bu1r, Wu2, bu2r,
      W_dec, b_decr)
    return (loss[0, 0], out.reshape(_NB, _NS, _DIM))


# speedup vs baseline: 3.1692x; 1.0240x over previous
import jax
import jax.numpy as jnp
from jax import lax
from jax.experimental import pallas as pl
from jax.experimental.pallas import tpu as pltpu

_DIM = 4
_NHID = 128
_NLAYERS = 3
_RADIUS = 0.35
_N = 4096
_NB = 8
_BT = 128
_NT = _N // _BT
_DSUB = 32
_NSUB = _BT // _DSUB
_NS = _N // _NB


def _mm(a, b):
    return lax.dot_general(a, b, (((1,), (1,)), ((), ())),
                           preferred_element_type=jnp.float32)


def _mmn(a, b):
    return lax.dot_general(a, b, (((1,), (0,)), ((), ())),
                           preferred_element_type=jnp.float32)


def _net_body(lob_ref, hib_ref,
              X_ref, batchc_ref, batch2d_ref, W_enc_ref, b_enc_ref,
              W1a_ref, W1b_ref, bm1_ref, Wm2_ref, bm2_ref,
              Wu1a_ref, Wu1b_ref, bu1_ref, Wu2_ref, bu2_ref,
              W_dec_ref, b_dec_ref,
              out_ref, loss_ref,
              h_ref, A_ref, B_ref, h2_ref, sqc_ref, sqm_ref,
              mask_ref, agg_ref, s1_ref, s2_ref, cnt_ref):
    f32 = jnp.float32
    r2 = jnp.float32(_RADIUS * _RADIUS)

    def enc_tile(t, c):
        xt = X_ref[pl.ds(t * _BT, _BT), :]
        sq = jnp.sum(xt * xt, axis=1, keepdims=True)
        sqc_ref[pl.ds(t * _BT, _BT), :] = sq
        h_ref[pl.ds(t * _BT, _BT), :] = _mm(xt, W_enc_ref[...]) + b_enc_ref[...]
        return c
    lax.fori_loop(0, _NT, enc_tile, 0)
    sqm_ref[...] = sqc_ref[...].reshape(_NT, _BT)

    iota_g = lax.broadcasted_iota(jnp.int32, (_NB, 1), 0)
    iota_gr = lax.broadcasted_iota(jnp.int32, (1, _NB), 1)
    eye_sub = (lax.broadcasted_iota(jnp.int32, (_DSUB, _DSUB), 0)
               == lax.broadcasted_iota(jnp.int32, (_DSUB, _DSUB), 1)
               ).astype(f32)

    for l in range(_NLAYERS):
        W2 = Wm2_ref[l]
        b2 = bm2_ref[l]
        W1a = W1a_ref[l]
        W1b = W1b_ref[l]
        b1 = bm1_ref[l]
        Wu1a = Wu1a_ref[l]
        Wu1b = Wu1b_ref[l]
        bu1 = bu1_ref[l]
        Wu2 = Wu2_ref[l]
        bu2 = bu2_ref[l]

        def ab_tile(t, c, W1a=W1a, W1b=W1b, b1=b1):
            ht = h_ref[pl.ds(t * _BT, _BT), :]
            A_ref[pl.ds(t * _BT, _BT), :] = _mm(ht, W1a) + b1
            B_ref[pl.ds(t * _BT, _BT), :] = _mm(ht, W1b)
            return c
        lax.fori_loop(0, _NT, ab_tile, 0)

        def dst_tile(t, carry, W2=W2, b2=b2, Wu1a=Wu1a, Wu1b=Wu1b,
                     bu1=bu1, Wu2=Wu2, bu2=bu2):
            d0 = t * _BT
            lob = lob_ref[t]
            hib = hib_ref[t]
            xd = X_ref[pl.ds(d0, _BT), :]
            sqd = sqc_ref[pl.ds(d0, _BT), :]
            bd = batchc_ref[pl.ds(d0, _BT), :]
            agg_ref[...] = jnp.zeros((_BT, _NHID), f32)

            def src_blk(s, c2, W2=W2, b2=b2, xd=xd, sqd=sqd, bd=bd, d0=d0):
                s0 = s * _BT
                xs = X_ref[pl.ds(s0, _BT), :]
                d2 = sqd + sqm_ref[pl.ds(s, 1), :] - 2.0 * _mm(xd, xs)
                bs = batch2d_ref[pl.ds(s, 1), :]
                mask_ref[...] = ((d2 <= r2) & (bd == bs)).astype(f32)
                bsrc = B_ref[pl.ds(s0, _BT), :]

                def strip(k, bsrc=bsrc, W2=W2, b2=b2, d0=d0):
                    r0 = k * _DSUB
                    a_sub = A_ref[pl.ds(d0 + r0, _DSUB), :]
                    m1 = jnp.maximum(
                        (a_sub[:, None, :] + bsrc[None, :, :]
                         ).astype(jnp.bfloat16),
                        jnp.bfloat16(0.0))
                    m2 = jnp.maximum(
                        _mm(m1.reshape(_DSUB * _BT, _NHID), W2) + b2, 0.0)
                    mks = mask_ref[pl.ds(r0, _DSUB), :]
                    m3 = m2.reshape(_DSUB, _BT, _NHID) * mks[:, :, None]
                    part = jnp.sum(m3, axis=1)
                    agg_ref[pl.ds(r0, _DSUB), :] = (
                        agg_ref[pl.ds(r0, _DSUB), :] + part)

                for k in range(_NSUB):
                    strip(k)
                return c2
            lax.fori_loop(lob, hib, src_blk, 0)

            agg = agg_ref[...]
            hd = h_ref[pl.ds(d0, _BT), :]
            u = jnp.maximum(_mm(hd, Wu1a) + _mm(agg, Wu1b) + bu1, 0.0)
            h2_ref[pl.ds(d0, _BT), :] = jnp.maximum(_mm(u, Wu2) + bu2, 0.0)
            return carry
        lax.fori_loop(0, _NT, dst_tile, 0)

        s1_ref[...] = jnp.zeros((_NB, _NHID), f32)
        s2_ref[...] = jnp.zeros((_NB, _NHID), f32)
        cnt_ref[...] = jnp.zeros((_NB, 1), f32)

        def stat_tile(t, c):
            h2t = h2_ref[pl.ds(t * _BT, _BT), :]
            brow = batch2d_ref[pl.ds(t, 1), :]
            oh = (iota_g == brow).astype(f32)
            s1_ref[...] = s1_ref[...] + _mmn(oh, h2t)
            s2_ref[...] = s2_ref[...] + _mmn(oh, h2t * h2t)
            cnt_ref[...] = cnt_ref[...] + jnp.sum(oh, axis=1, keepdims=True)
            return c
        lax.fori_loop(0, _NT, stat_tile, 0)

        cnt = cnt_ref[...]
        mu = s1_ref[...] / cnt
        va = s2_ref[...] / cnt - mu * mu

        def norm_tile(t, c, mu=mu, va=va):
            h2t = h2_ref[pl.ds(t * _BT, _BT), :]
            bcol = batchc_ref[pl.ds(t * _BT, _BT), :]
            ohc = (bcol == iota_gr).astype(f32)
            mug = _mmn(ohc, mu)
            vag = _mmn(ohc, va)
            h_ref[pl.ds(t * _BT, _BT), :] = (h2t - mug) / jnp.sqrt(vag + 1e-5)
            return c
        lax.fori_loop(0, _NT, norm_tile, 0)

    def dec_tile(t, c):
        ht = h_ref[pl.ds(t * _BT, _BT), :]
        out_ref[pl.ds(t * _BT, _BT), :] = jax.nn.sigmoid(
            _mm(ht, W_dec_ref[...]) + b_dec_ref[...])
        return c
    lax.fori_loop(0, _NT, dec_tile, 0)

    iota_d = lax.broadcasted_iota(jnp.int32, (1, _DIM), 1)

    def disc_graph(g, total):
        x = out_ref[pl.ds(g * _NS, _NS), :]
        om = 1.0 - x * x
        p1 = om[:, 0:1] * om[:, 1:2] * om[:, 2:3] * om[:, 3:4]
        sum1 = jnp.sum(p1)
        accp = jnp.ones((_NS, _NS), jnp.float32)
        for d in range(_DIM):
            ed = (iota_d == d).astype(jnp.float32)
            row_d = _mm(ed, x)
            col_d = x[:, d:d + 1]
            accp = accp * (1.0 - jnp.maximum(col_d, row_d))
        sum2 = jnp.sum(accp)
        disc = jnp.sqrt(3.0 ** (-_DIM)
                        - (1.0 / _NS) * (2.0 ** (1 - _DIM)) * sum1
                        + (1.0 / (_NS * _NS)) * sum2)
        return total + disc
    total = lax.fori_loop(0, _NB, disc_graph,
                          jnp.zeros((1, 1), jnp.float32))
    loss_ref[...] = total / _NB


def kernel(X, batch, W_enc, b_enc, Wm1, bm1, Wm2, bm2, Wu1, bu1, Wu2, bu2,
           W_dec, b_dec):
    f32 = jnp.float32
    batch = batch.astype(jnp.int32)
    batchc = batch.reshape(_N, 1)
    batch2d = batch.reshape(_NT, _BT)
    bmin = batch2d[:, 0]
    bmax = batch2d[:, -1]
    lo = jnp.searchsorted(batch, bmin, side="left").astype(jnp.int32)
    hi = jnp.searchsorted(batch, bmax, side="right").astype(jnp.int32)
    lob = lo // _BT
    hib = (hi + _BT - 1) // _BT

    W1a = Wm1[:, :, :_NHID]
    W1b = Wm1[:, :, _NHID:]
    Wm2b = Wm2.astype(jnp.bfloat16)
    Wu1a = Wu1[:, :, :_NHID]
    Wu1b = Wu1[:, :, _NHID:]
    bm1r = bm1.reshape(_NLAYERS, 1, _NHID)
    bu1r = bu1.reshape(_NLAYERS, 1, _NHID)
    bm2r = bm2.reshape(_NLAYERS, 1, _NHID)
    bu2r = bu2.reshape(_NLAYERS, 1, _NHID)
    b_encr = b_enc.reshape(1, _NHID)
    b_decr = b_dec.reshape(1, _DIM)

    smem = pl.BlockSpec(memory_space=pltpu.MemorySpace.SMEM)
    vmem = pl.BlockSpec(memory_space=pltpu.MemorySpace.VMEM)

    out, loss = pl.pallas_call(
        _net_body,
        in_specs=[smem, smem] + [vmem] * 17,
        out_shape=[
            jax.ShapeDtypeStruct((_N, _DIM), f32),
            jax.ShapeDtypeStruct((1, 1), f32),
        ],
        scratch_shapes=[
            pltpu.VMEM((_N, _NHID), f32),
            pltpu.VMEM((_N, _NHID), f32),
            pltpu.VMEM((_N, _NHID), f32),
            pltpu.VMEM((_N, _NHID), f32),
            pltpu.VMEM((_N, 1), f32),
            pltpu.VMEM((_NT, _BT), f32),
            pltpu.VMEM((_BT, _BT), f32),
            pltpu.VMEM((_BT, _NHID), f32),
            pltpu.VMEM((_NB, _NHID), f32),
            pltpu.VMEM((_NB, _NHID), f32),
            pltpu.VMEM((_NB, 1), f32),
        ],
    )(lob, hib,
      X, batchc, batch2d, W_enc, b_encr,
      W1a, W1b, bm1r, Wm2b, bm2r,
      Wu1a, Wu1b, bu1r, Wu2, bu2r,
      W_dec, b_decr)
    return (loss[0, 0], out.reshape(_NB, _NS, _DIM))
